# baseline (speedup 1.0000x reference)
import functools

import jax
import jax.numpy as jnp
from jax import lax
from jax.experimental import pallas as pl
from jax.experimental.pallas import tpu as pltpu
from jax.experimental.pallas import tpu_sc as plsc

_VOCAB = 100000
_EMBED = 64
_CTX = 200
_HIDDEN = 128

_BV = 2048
_NB = -(-_VOCAB // _BV)
_PADV = _NB * _BV


def _sc_embed_sum(idx, table):
    mesh = plsc.VectorSubcoreMesh(core_axis_name="c", subcore_axis_name="s")

    @functools.partial(
        pl.kernel,
        out_type=jax.ShapeDtypeStruct((_EMBED,), jnp.float32),
        mesh=mesh,
        scratch_types=[
            pltpu.VMEM((_CTX,), jnp.int32),
            pltpu.VMEM((_CTX, _EMBED), jnp.float32),
            pltpu.VMEM((_EMBED,), jnp.float32),
            pltpu.SemaphoreType.DMA,
        ],
        compiler_params=pltpu.CompilerParams(use_tc_tiling_on_sc=False),
    )
    def k(idx_hbm, table_hbm, out_hbm, idx_v, rows_v, acc_v, sem):
        cid = lax.axis_index("c")
        sid = lax.axis_index("s")
        wid = sid * 2 + cid

        @pl.when(wid == 0)
        def _():
            pltpu.sync_copy(idx_hbm, idx_v)
            pltpu.async_copy(table_hbm.at[idx_v], rows_v, sem).wait()

            def body(r, carry):
                return tuple(
                    c + rows_v[r, pl.ds(16 * i, 16)] for i, c in enumerate(carry)
                )

            z = jnp.zeros((16,), jnp.float32)
            acc = lax.fori_loop(0, _CTX, body, (z, z, z, z))
            for i in range(4):
                acc_v[pl.ds(16 * i, 16)] = acc[i]
            pltpu.sync_copy(acc_v, out_hbm)

    return k(idx, table)


def _tc_body(emb_ref, w1_ref, b1_ref, w2_ref, b2_ref, out_ref, h_s, out_s):
    j = pl.program_id(0)

    @pl.when(j == 0)
    def _():
        h_s[...] = jnp.maximum(
            jax.lax.dot_general(
                emb_ref[...], w1_ref[...], (((1,), (0,)), ((), ())),
                preferred_element_type=jnp.float32,
            ) + b1_ref[...],
            0.0,
        )

    h = h_s[...]
    blk = jax.lax.dot_general(
        h, w2_ref[...], (((1,), (0,)), ((), ())),
        preferred_element_type=jnp.float32,
    ) + b2_ref[...]
    out_s[:, pl.ds(j * _BV, _BV)] = blk

    @pl.when(j == _NB - 1)
    def _():
        full = out_s[...]
        col = lax.broadcasted_iota(jnp.int32, (1, _PADV), 1)
        valid = col < _VOCAB
        m = jnp.max(jnp.where(valid, full, -jnp.inf))
        e = jnp.where(valid, jnp.exp(full - m), 0.0)
        lse = m + jnp.log(jnp.sum(e))
        out_ref[...] = (full - lse)[:, :_VOCAB]


def _tc_mlp(embeds, W1, b1, W2, b2):
    return pl.pallas_call(
        _tc_body,
        grid=(_NB,),
        in_specs=[
            pl.BlockSpec((1, _EMBED), lambda j: (0, 0)),
            pl.BlockSpec((_EMBED, _HIDDEN), lambda j: (0, 0)),
            pl.BlockSpec((1, _HIDDEN), lambda j: (0, 0)),
            pl.BlockSpec((_HIDDEN, _BV), lambda j: (0, j)),
            pl.BlockSpec((1, _BV), lambda j: (0, j)),
        ],
        out_specs=pl.BlockSpec((1, _VOCAB), lambda j: (0, 0)),
        out_shape=jax.ShapeDtypeStruct((1, _VOCAB), jnp.float32),
        scratch_shapes=[
            pltpu.VMEM((1, _HIDDEN), jnp.float32),
            pltpu.VMEM((1, _PADV), jnp.float32),
        ],
        compiler_params=pltpu.CompilerParams(
            dimension_semantics=("arbitrary",),
        ),
    )(embeds.reshape(1, _EMBED), W1, b1.reshape(1, _HIDDEN), W2,
      b2.reshape(1, _VOCAB))


def kernel(inputs, emb_table, W1, b1, W2, b2):
    idx = inputs.astype(jnp.int32)
    embeds = _sc_embed_sum(idx, emb_table)
    log_probs = _tc_mlp(embeds, W1, b1, W2, b2)
    return (log_probs, embeds)


# speedup vs baseline: 1.1435x; 1.1435x over previous
import functools

import jax
import jax.numpy as jnp
from jax import lax
from jax.experimental import pallas as pl
from jax.experimental.pallas import tpu as pltpu
from jax.experimental.pallas import tpu_sc as plsc

_VOCAB = 100000
_EMBED = 64
_CTX = 200
_HIDDEN = 128

_CB = 8192
_NFULL = _VOCAB // _CB
_TAIL = _VOCAB - _NFULL * _CB
_PADV = (_NFULL + 1) * _CB
_NBUF = 4


def _sc_embed_sum(idx, table):
    mesh = plsc.VectorSubcoreMesh(core_axis_name="c", subcore_axis_name="s")

    @functools.partial(
        pl.kernel,
        out_type=jax.ShapeDtypeStruct((_EMBED,), jnp.float32),
        mesh=mesh,
        scratch_types=[
            pltpu.VMEM((_CTX,), jnp.int32),
            pltpu.VMEM((_CTX, _EMBED), jnp.float32),
            pltpu.VMEM((_EMBED,), jnp.float32),
            pltpu.SemaphoreType.DMA,
        ],
        compiler_params=pltpu.CompilerParams(use_tc_tiling_on_sc=False),
    )
    def k(idx_hbm, table_hbm, out_hbm, idx_v, rows_v, acc_v, sem):
        cid = lax.axis_index("c")
        sid = lax.axis_index("s")
        wid = sid * 2 + cid

        @pl.when(wid == 0)
        def _():
            pltpu.sync_copy(idx_hbm, idx_v)
            pltpu.async_copy(table_hbm.at[idx_v], rows_v, sem).wait()

            def body(r, carry):
                return tuple(
                    c + rows_v[r, pl.ds(16 * i, 16)] for i, c in enumerate(carry)
                )

            z = jnp.zeros((16,), jnp.float32)
            acc = lax.fori_loop(0, _CTX, body, (z, z, z, z))
            for i in range(4):
                acc_v[pl.ds(16 * i, 16)] = acc[i]
            pltpu.sync_copy(acc_v, out_hbm)

    return k(idx, table)


def _tc_body(emb_ref, w1_ref, b1_ref, b2_ref, w2_hbm, out_ref,
             bufs, tail_buf, out_s, sems, tail_sem):
    h = jnp.maximum(
        jax.lax.dot_general(
            emb_ref[...], w1_ref[...], (((1,), (0,)), ((), ())),
            preferred_element_type=jnp.float32,
        ) + b1_ref[...],
        0.0,
    )

    def chunk_copy(c, slot):
        return pltpu.make_async_copy(
            w2_hbm.at[:, pl.ds(c * _CB, _CB)], bufs.at[slot], sems.at[slot]
        )

    for k in range(_NBUF):
        chunk_copy(k, k).start()
    tail_cp = pltpu.make_async_copy(
        w2_hbm.at[:, pl.ds(_NFULL * _CB, _TAIL)], tail_buf, tail_sem
    )
    tail_cp.start()

    for c in range(_NFULL):
        slot = c % _NBUF
        chunk_copy(c, slot).wait()
        blk = jax.lax.dot_general(
            h, bufs[slot], (((1,), (0,)), ((), ())),
            preferred_element_type=jnp.float32,
        ) + b2_ref[:, pl.ds(c * _CB, _CB)]
        out_s[:, pl.ds(c * _CB, _CB)] = blk
        nxt = c + _NBUF
        if nxt < _NFULL:
            chunk_copy(nxt, slot).start()

    tail_cp.wait()
    blk_t = jax.lax.dot_general(
        h, tail_buf[...], (((1,), (0,)), ((), ())),
        preferred_element_type=jnp.float32,
    ) + b2_ref[:, pl.ds(_NFULL * _CB, _TAIL)]
    out_s[:, pl.ds(_NFULL * _CB, _TAIL)] = blk_t

    full = out_s[...]
    col = lax.broadcasted_iota(jnp.int32, (1, _PADV), 1)
    valid = col < _VOCAB
    m = jnp.max(jnp.where(valid, full, -jnp.inf))
    e = jnp.where(valid, jnp.exp(full - m), 0.0)
    lse = m + jnp.log(jnp.sum(e))
    out_ref[...] = (full - lse)[:, :_VOCAB]


def _tc_mlp(embeds, W1, b1, W2, b2):
    return pl.pallas_call(
        _tc_body,
        in_specs=[
            pl.BlockSpec((1, _EMBED), lambda: (0, 0)),
            pl.BlockSpec((_EMBED, _HIDDEN), lambda: (0, 0)),
            pl.BlockSpec((1, _HIDDEN), lambda: (0, 0)),
            pl.BlockSpec((1, _VOCAB), lambda: (0, 0)),
            pl.BlockSpec(memory_space=pl.ANY),
        ],
        out_specs=pl.BlockSpec((1, _VOCAB), lambda: (0, 0)),
        out_shape=jax.ShapeDtypeStruct((1, _VOCAB), jnp.float32),
        scratch_shapes=[
            pltpu.VMEM((_NBUF, _HIDDEN, _CB), jnp.float32),
            pltpu.VMEM((_HIDDEN, _TAIL), jnp.float32),
            pltpu.VMEM((1, _PADV), jnp.float32),
            pltpu.SemaphoreType.DMA((_NBUF,)),
            pltpu.SemaphoreType.DMA,
        ],
    )(embeds.reshape(1, _EMBED), W1, b1.reshape(1, _HIDDEN),
      b2.reshape(1, _VOCAB), W2)


def kernel(inputs, emb_table, W1, b1, W2, b2):
    idx = inputs.astype(jnp.int32)
    embeds = _sc_embed_sum(idx, emb_table)
    log_probs = _tc_mlp(embeds, W1, b1, W2, b2)
    return (log_probs, embeds)


# speedup vs baseline: 3.7028x; 3.2382x over previous
import jax
import jax.numpy as jnp
from jax import lax
from jax.experimental import pallas as pl
from jax.experimental.pallas import tpu as pltpu

_VOCAB = 100000
_EMBED = 64
_CTX = 200
_HIDDEN = 128

_CB = 8192
_NFULL = _VOCAB // _CB
_TAIL = _VOCAB - _NFULL * _CB
_PADV = (_NFULL + 1) * _CB
_NBUF = 4
_NWIN = 8


def _body(idx_ref, w1_ref, b1_ref, b2_ref, tab_hbm, w2t_hbm,
          out_ref, emb_ref, wins, bufs, tail_buf, out_s, acc_s,
          win_sems, sems, tail_sem):
    def chunk_copy(c, slot):
        return pltpu.make_async_copy(
            w2t_hbm.at[pl.ds(c * _CB, _CB), :], bufs.at[slot], sems.at[slot]
        )

    for k in range(_NBUF):
        chunk_copy(k, k).start()
    tail_cp = pltpu.make_async_copy(
        w2t_hbm.at[pl.ds(_NFULL * _CB, _TAIL), :], tail_buf, tail_sem
    )
    tail_cp.start()

    def win_copy(r, slot):
        i = idx_ref[r]
        c0 = pl.multiple_of((i // 128) * 128, 128)
        return pltpu.make_async_copy(
            tab_hbm.at[:, pl.ds(c0, 128)], wins.at[slot], win_sems.at[slot]
        )

    for r in range(_NWIN):
        win_copy(r, r).start()

    acc_s[...] = jnp.zeros((_EMBED, 128), jnp.float32)
    lane = lax.broadcasted_iota(jnp.int32, (_EMBED, 128), 1)
    for r in range(_CTX):
        slot = r % _NWIN
        win_copy(r, slot).wait()
        i = idx_ref[r]
        acc_s[...] += jnp.where(lane == (i % 128), wins[slot], 0.0)
        nxt = r + _NWIN
        if nxt < _CTX:
            win_copy(nxt, slot).start()

    embeds = jnp.sum(acc_s[...], axis=1)
    emb_ref[...] = embeds

    h = jnp.maximum(
        jax.lax.dot_general(
            embeds.reshape(1, _EMBED), w1_ref[...], (((1,), (0,)), ((), ())),
            preferred_element_type=jnp.float32,
        ) + b1_ref[...],
        0.0,
    )

    for c in range(_NFULL):
        slot = c % _NBUF
        chunk_copy(c, slot).wait()
        blk = jax.lax.dot_general(
            h, bufs[slot], (((1,), (1,)), ((), ())),
            preferred_element_type=jnp.float32,
        ) + b2_ref[:, pl.ds(c * _CB, _CB)]
        out_s[:, pl.ds(c * _CB, _CB)] = blk
        nxt = c + _NBUF
        if nxt < _NFULL:
            chunk_copy(nxt, slot).start()

    tail_cp.wait()
    blk_t = jax.lax.dot_general(
        h, tail_buf[...], (((1,), (1,)), ((), ())),
        preferred_element_type=jnp.float32,
    ) + b2_ref[:, pl.ds(_NFULL * _CB, _TAIL)]
    out_s[:, pl.ds(_NFULL * _CB, _TAIL)] = blk_t

    full = out_s[...]
    col = lax.broadcasted_iota(jnp.int32, (1, _PADV), 1)
    valid = col < _VOCAB
    m = jnp.max(jnp.where(valid, full, -jnp.inf))
    e = jnp.where(valid, jnp.exp(full - m), 0.0)
    lse = m + jnp.log(jnp.sum(e))
    out_ref[...] = (full - lse)[:, :_VOCAB]


def _fused(idx, tab_t, W1, b1, w2t, b2):
    return pl.pallas_call(
        _body,
        in_specs=[
            pl.BlockSpec(memory_space=pltpu.SMEM),
            pl.BlockSpec((_EMBED, _HIDDEN), lambda: (0, 0)),
            pl.BlockSpec((1, _HIDDEN), lambda: (0, 0)),
            pl.BlockSpec((1, _VOCAB), lambda: (0, 0)),
            pl.BlockSpec(memory_space=pl.ANY),
            pl.BlockSpec(memory_space=pl.ANY),
        ],
        out_specs=[
            pl.BlockSpec((1, _VOCAB), lambda: (0, 0)),
            pl.BlockSpec(memory_space=pltpu.VMEM),
        ],
        out_shape=[
            jax.ShapeDtypeStruct((1, _VOCAB), jnp.float32),
            jax.ShapeDtypeStruct((_EMBED,), jnp.float32),
        ],
        scratch_shapes=[
            pltpu.VMEM((_NWIN, _EMBED, 128), jnp.float32),
            pltpu.VMEM((_NBUF, _CB, _HIDDEN), jnp.float32),
            pltpu.VMEM((_TAIL, _HIDDEN), jnp.float32),
            pltpu.VMEM((1, _PADV), jnp.float32),
            pltpu.VMEM((_EMBED, 128), jnp.float32),
            pltpu.SemaphoreType.DMA((_NWIN,)),
            pltpu.SemaphoreType.DMA((_NBUF,)),
            pltpu.SemaphoreType.DMA,
        ],
    )(idx, W1, b1.reshape(1, _HIDDEN), b2.reshape(1, _VOCAB), tab_t, w2t)


def kernel(inputs, emb_table, W1, b1, W2, b2):
    idx = inputs.astype(jnp.int32)
    log_probs, embeds = _fused(idx, emb_table.T, W1, b1, W2.T, b2)
    return (log_probs, embeds)


# speedup vs baseline: 4.6964x; 1.2683x over previous
import jax
import jax.numpy as jnp
from jax import lax
from jax.experimental import pallas as pl
from jax.experimental.pallas import tpu as pltpu

_VOCAB = 100000
_EMBED = 64
_CTX = 200
_HIDDEN = 128

_CB = 8192
_NFULL = _VOCAB // _CB
_TAIL = _VOCAB - _NFULL * _CB
_PADV = (_NFULL + 1) * _CB
_NBUF = 8
_NWIN = 16


def _body(idx_ref, w1_ref, b1_ref, b2_ref, tab_hbm, w2t_hbm,
          out_ref, emb_ref, wins, bufs, tail_buf, out_s, acc_s,
          win_sems, sems, tail_sem):
    def chunk_copy(c, slot):
        return pltpu.make_async_copy(
            w2t_hbm.at[pl.ds(c * _CB, _CB), :], bufs.at[slot], sems.at[slot]
        )

    for k in range(_NBUF):
        chunk_copy(k, k).start()
    tail_cp = pltpu.make_async_copy(
        w2t_hbm.at[pl.ds(_NFULL * _CB, _TAIL), :], tail_buf, tail_sem
    )
    tail_cp.start()

    def win_copy(r, slot):
        i = idx_ref[r]
        c0 = pl.multiple_of((i // 128) * 128, 128)
        return pltpu.make_async_copy(
            tab_hbm.at[:, pl.ds(c0, 128)], wins.at[slot], win_sems.at[slot]
        )

    for r in range(_NWIN):
        win_copy(r, r).start()

    acc_s[...] = jnp.zeros((_EMBED, 128), jnp.float32)
    lane = lax.broadcasted_iota(jnp.int32, (_EMBED, 128), 1)
    for r in range(_CTX):
        slot = r % _NWIN
        win_copy(r, slot).wait()
        i = idx_ref[r]
        acc_s[...] += jnp.where(lane == (i % 128), wins[slot], 0.0)
        nxt = r + _NWIN
        if nxt < _CTX:
            win_copy(nxt, slot).start()

    embeds = jnp.sum(acc_s[...], axis=1)
    emb_ref[...] = embeds

    h = jnp.maximum(
        jax.lax.dot_general(
            embeds.reshape(1, _EMBED), w1_ref[...], (((1,), (0,)), ((), ())),
            preferred_element_type=jnp.float32,
        ) + b1_ref[...],
        0.0,
    )

    for c in range(_NFULL):
        slot = c % _NBUF
        chunk_copy(c, slot).wait()
        blk = jax.lax.dot_general(
            h, bufs[slot], (((1,), (1,)), ((), ())),
            preferred_element_type=jnp.float32,
        ) + b2_ref[:, pl.ds(c * _CB, _CB)]
        out_s[:, pl.ds(c * _CB, _CB)] = blk
        nxt = c + _NBUF
        if nxt < _NFULL:
            chunk_copy(nxt, slot).start()

    tail_cp.wait()
    blk_t = jax.lax.dot_general(
        h, tail_buf[...], (((1,), (1,)), ((), ())),
        preferred_element_type=jnp.float32,
    ) + b2_ref[:, pl.ds(_NFULL * _CB, _TAIL)]
    out_s[:, pl.ds(_NFULL * _CB, _TAIL)] = blk_t

    full = out_s[...]
    col = lax.broadcasted_iota(jnp.int32, (1, _PADV), 1)
    valid = col < _VOCAB
    m = jnp.max(jnp.where(valid, full, -jnp.inf))
    e = jnp.where(valid, jnp.exp(full - m), 0.0)
    lse = m + jnp.log(jnp.sum(e))
    out_ref[...] = (full - lse)[:, :_VOCAB]


def _fused(idx, tab_t, W1, b1, w2t, b2):
    return pl.pallas_call(
        _body,
        in_specs=[
            pl.BlockSpec(memory_space=pltpu.SMEM),
            pl.BlockSpec((_EMBED, _HIDDEN), lambda: (0, 0)),
            pl.BlockSpec((1, _HIDDEN), lambda: (0, 0)),
            pl.BlockSpec((1, _VOCAB), lambda: (0, 0)),
            pl.BlockSpec(memory_space=pl.ANY),
            pl.BlockSpec(memory_space=pl.ANY),
        ],
        out_specs=[
            pl.BlockSpec((1, _VOCAB), lambda: (0, 0)),
            pl.BlockSpec(memory_space=pltpu.VMEM),
        ],
        out_shape=[
            jax.ShapeDtypeStruct((1, _VOCAB), jnp.float32),
            jax.ShapeDtypeStruct((_EMBED,), jnp.float32),
        ],
        scratch_shapes=[
            pltpu.VMEM((_NWIN, _EMBED, 128), jnp.float32),
            pltpu.VMEM((_NBUF, _CB, _HIDDEN), jnp.float32),
            pltpu.VMEM((_TAIL, _HIDDEN), jnp.float32),
            pltpu.VMEM((1, _PADV), jnp.float32),
            pltpu.VMEM((_EMBED, 128), jnp.float32),
            pltpu.SemaphoreType.DMA((_NWIN,)),
            pltpu.SemaphoreType.DMA((_NBUF,)),
            pltpu.SemaphoreType.DMA,
        ],
    )(idx, W1, b1.reshape(1, _HIDDEN), b2.reshape(1, _VOCAB), tab_t, w2t)


def kernel(inputs, emb_table, W1, b1, W2, b2):
    idx = inputs.astype(jnp.int32)
    log_probs, embeds = _fused(idx, emb_table.T, W1, b1, W2.T, b2)
    return (log_probs, embeds)
